# Initial kernel scaffold; baseline (speedup 1.0000x reference)
import jax
import jax.numpy as jnp
from jax.experimental import pallas as pl


def kernel(x, embeddings):
    raise NotImplementedError("write your pallas kernel here")


# speedup vs baseline: 2.1631x; 2.1631x over previous
import jax
import jax.numpy as jnp
from jax.experimental import pallas as pl
from jax.experimental.pallas import tpu as pltpu

_B, _D, _T = 32, 64, 576
_K = 1024
_COMMITMENT_COST = 0.25
_VQ_COST = 1.0


def _vq_kernel(x_ref, e_ref, q_ref, idx_ref, sse_ref):
    b = pl.program_id(0)
    xb = x_ref[0]
    emb = e_ref[...]
    g = jax.lax.dot_general(emb, xb, (((1,), (0,)), ((), ())),
                            preferred_element_type=jnp.float32)
    e_norm = jnp.sum(emb * emb, axis=1, keepdims=True)
    score = e_norm - 2.0 * g
    minv = jnp.min(score, axis=0)
    iota_k = jax.lax.broadcasted_iota(jnp.int32, (_K, _T), 0)
    idx = jnp.min(jnp.where(score == minv[None, :], iota_k, _K), axis=0)
    idx_ref[0, 0] = idx
    onehot = (iota_k == idx[None, :]).astype(jnp.float32)
    qb = jax.lax.dot_general(emb, onehot, (((0,), (0,)), ((), ())),
                             preferred_element_type=jnp.float32)
    q_ref[0] = qb
    diff = xb - qb
    sse = jnp.sum(diff * diff).reshape(1, 1)

    @pl.when(b == 0)
    def _init():
        sse_ref[...] = jnp.zeros((1, 1), jnp.float32)

    sse_ref[...] += sse


def kernel(x, embeddings):
    q, idx, sse = pl.pallas_call(
        _vq_kernel,
        grid=(_B,),
        in_specs=[
            pl.BlockSpec((1, _D, _T), lambda b: (b, 0, 0)),
            pl.BlockSpec((_K, _D), lambda b: (0, 0)),
        ],
        out_specs=[
            pl.BlockSpec((1, _D, _T), lambda b: (b, 0, 0)),
            pl.BlockSpec((1, 1, _T), lambda b: (b, 0, 0)),
            pl.BlockSpec((1, 1), lambda b: (0, 0)),
        ],
        out_shape=[
            jax.ShapeDtypeStruct((_B, _D, _T), jnp.float32),
            jax.ShapeDtypeStruct((_B, 1, _T), jnp.int32),
            jax.ShapeDtypeStruct((1, 1), jnp.float32),
        ],
    )(x, embeddings)
    e = sse[0, 0] / (_B * _T * _D)
    loss_commit = _COMMITMENT_COST * e
    loss_vq = _VQ_COST * e
    return q, loss_commit, loss_vq, idx.reshape(_B * _T)


# speedup vs baseline: 2.6701x; 1.2344x over previous
import jax
import jax.numpy as jnp
from jax.experimental import pallas as pl
from jax.experimental.pallas import tpu as pltpu

_B, _D, _T = 32, 64, 576
_K = 1024
_COMMITMENT_COST = 0.25
_VQ_COST = 1.0


def _vq_kernel(x_ref, e_ref, q_ref, idx_ref, sse_ref):
    xb = x_ref[0]
    emb = e_ref[...]
    g = jax.lax.dot_general(emb, xb, (((1,), (0,)), ((), ())),
                            preferred_element_type=jnp.float32)
    e_norm = jnp.sum(emb * emb, axis=1, keepdims=True)
    score = e_norm - 2.0 * g
    idx = jnp.argmin(score, axis=0).astype(jnp.int32)
    idx_ref[0, 0] = idx
    iota_k = jax.lax.broadcasted_iota(jnp.int32, (_K, _T), 0)
    onehot = (iota_k == idx[None, :]).astype(jnp.float32)
    qb = jax.lax.dot_general(emb, onehot, (((0,), (0,)), ((), ())),
                             preferred_element_type=jnp.float32)
    q_ref[0] = qb
    diff = xb - qb
    sse_ref[...] = jnp.sum(diff * diff).reshape(1, 1, 1)


def kernel(x, embeddings):
    q, idx, sse = pl.pallas_call(
        _vq_kernel,
        grid=(_B,),
        in_specs=[
            pl.BlockSpec((1, _D, _T), lambda b: (b, 0, 0)),
            pl.BlockSpec((_K, _D), lambda b: (0, 0)),
        ],
        out_specs=[
            pl.BlockSpec((1, _D, _T), lambda b: (b, 0, 0)),
            pl.BlockSpec((1, 1, _T), lambda b: (b, 0, 0)),
            pl.BlockSpec((1, 1, 1), lambda b: (b, 0, 0)),
        ],
        out_shape=[
            jax.ShapeDtypeStruct((_B, _D, _T), jnp.float32),
            jax.ShapeDtypeStruct((_B, 1, _T), jnp.int32),
            jax.ShapeDtypeStruct((_B, 1, 1), jnp.float32),
        ],
        compiler_params=pltpu.CompilerParams(
            dimension_semantics=("parallel",),
        ),
    )(x, embeddings)
    e = jnp.sum(sse) / (_B * _T * _D)
    loss_commit = _COMMITMENT_COST * e
    loss_vq = _VQ_COST * e
    return q, loss_commit, loss_vq, idx.reshape(_B * _T)
